# baseline (speedup 1.0000x reference)
import functools

import jax
import jax.numpy as jnp
from jax import lax
from jax.experimental import pallas as pl
from jax.experimental.pallas import tpu as pltpu
from jax.experimental.pallas import tpu_sc as plsc

NE_PAD = 128
LANES = 16
NCHUNK = 4


def _make_hist(B, L, num_cores, num_subcores):
    NW = num_cores * num_subcores
    rows = B // NW
    mesh = plsc.VectorSubcoreMesh(core_axis_name="c", subcore_axis_name="s")

    @functools.partial(
        pl.kernel,
        out_type=jax.ShapeDtypeStruct((B, NE_PAD), jnp.float32),
        mesh=mesh,
        compiler_params=pltpu.CompilerParams(needs_layout_passes=False),
        scratch_types=[
            pltpu.VMEM((2 * L, rows), jnp.int32),
            pltpu.VMEM((rows, NE_PAD), jnp.float32),
        ]
        + [pltpu.SemaphoreType.DMA] * (2 * NCHUNK),
    )
    def hist(packed_hbm, w_hbm, packed_v, w_v, *sems):
        in_sems, wb_sems = sems[:NCHUNK], sems[NCHUNK:]
        wid = lax.axis_index("s") * num_cores + lax.axis_index("c")
        base = wid * rows
        crows = rows // NCHUNK
        cgroups = crows // LANES
        in_cps = [
            pltpu.async_copy(
                packed_hbm.at[:, pl.ds(base + ch * crows, crows)],
                packed_v.at[:, pl.ds(ch * crows, crows)], in_sems[ch])
            for ch in range(NCHUNK)
        ]

        lane = lax.iota(jnp.int32, LANES)
        zeros = jnp.zeros((LANES,), jnp.float32)

        @plsc.parallel_loop(0, rows // 8)
        def _(z):
            for r in range(8):
                for j in range(NE_PAD // LANES):
                    w_v[z * 8 + r, pl.ds(j * LANES, LANES)] = zeros

        wb_cps = []
        for ch in range(NCHUNK):
            in_cps[ch].wait()

            @plsc.parallel_loop(0, cgroups, unroll=2)
            def _(g, ch=ch):
                row0 = ch * crows + g * LANES
                rowv = row0 + lane
                for l in range(L):
                    c = packed_v[l, pl.ds(row0, LANES)]
                    mbits = packed_v[L + l, pl.ds(row0, LANES)]
                    m = plsc.bitcast(mbits, jnp.float32)
                    plsc.addupdate_scatter(w_v, [rowv, c], m)

            wb_cps.append(
                pltpu.async_copy(w_v.at[pl.ds(ch * crows, crows), :],
                                 w_hbm.at[pl.ds(base + ch * crows, crows), :],
                                 wb_sems[ch]))
        for cp in wb_cps:
            cp.wait()

    return hist


def _mm_body(ne_pad, w_ref, e_ref, o_ref):
    ne = e_ref.shape[0]
    epad = jnp.pad(e_ref[...], ((0, ne_pad - ne), (0, 0)))
    o_ref[...] = jnp.dot(w_ref[...], epad,
                         preferred_element_type=jnp.float32)


def kernel(cards, mask, embedding):
    B, L = cards.shape
    NE, D = embedding.shape
    info = plsc.get_sparse_core_info()

    packed = jnp.concatenate(
        [cards.astype(jnp.int32).T,
         lax.bitcast_convert_type(mask, jnp.int32).T], axis=0)

    hist = _make_hist(B, L, info.num_cores, info.num_subcores)
    w = hist(packed)

    BM = 4096
    out = pl.pallas_call(
        functools.partial(_mm_body, NE_PAD),
        grid=(B // BM,),
        in_specs=[
            pl.BlockSpec((BM, NE_PAD), lambda i: (i, 0)),
            pl.BlockSpec((NE, D), lambda i: (0, 0)),
        ],
        out_specs=pl.BlockSpec((BM, D), lambda i: (i, 0)),
        out_shape=jax.ShapeDtypeStruct((B, D), jnp.float32),
    )(w, embedding)
    return out


# speedup vs baseline: 1.1484x; 1.1484x over previous
import functools

import jax
import jax.numpy as jnp
from jax import lax
from jax.experimental import pallas as pl
from jax.experimental.pallas import tpu as pltpu
from jax.experimental.pallas import tpu_sc as plsc

NE_PAD = 128
LANES = 16


def _make_hist(B, L, num_cores, num_subcores):
    NW = num_cores * num_subcores
    rows = B // NW
    mesh = plsc.VectorSubcoreMesh(core_axis_name="c", subcore_axis_name="s")

    @functools.partial(
        pl.kernel,
        out_type=jax.ShapeDtypeStruct((B, NE_PAD), jnp.float32),
        mesh=mesh,
        compiler_params=pltpu.CompilerParams(needs_layout_passes=False),
        scratch_types=[
            pltpu.VMEM((2 * L, rows), jnp.int32),
            pltpu.VMEM((rows, NE_PAD), jnp.float32),
            pltpu.SemaphoreType.DMA,
            pltpu.SemaphoreType.DMA,
        ],
    )
    def hist(packed_hbm, w_hbm, packed_v, w_v, sem0, sem1):
        wid = lax.axis_index("s") * num_cores + lax.axis_index("c")
        base = wid * rows
        half = rows // 2
        groups = rows // LANES
        cp0 = pltpu.async_copy(packed_hbm.at[:, pl.ds(base, half)],
                               packed_v.at[:, pl.ds(0, half)], sem0)
        cp1 = pltpu.async_copy(packed_hbm.at[:, pl.ds(base + half, half)],
                               packed_v.at[:, pl.ds(half, half)], sem1)

        lane = lax.iota(jnp.int32, LANES)
        zeros = jnp.zeros((LANES,), jnp.float32)

        @plsc.parallel_loop(0, rows // 8)
        def _(z):
            for r in range(8):
                for j in range(NE_PAD // LANES):
                    w_v[z * 8 + r, pl.ds(j * LANES, LANES)] = zeros

        cp0.wait()
        cp1.wait()

        @plsc.parallel_loop(0, groups, unroll=2)
        def _(g):
            row0 = g * LANES
            rowv = row0 + lane
            for l in range(L):
                c = packed_v[l, pl.ds(row0, LANES)]
                mbits = packed_v[L + l, pl.ds(row0, LANES)]
                m = plsc.bitcast(mbits, jnp.float32)
                plsc.addupdate_scatter(w_v, [rowv, c], m)

        pltpu.sync_copy(w_v, w_hbm.at[pl.ds(base, rows), :])

    return hist


def _mm_body(ne_pad, w_ref, e_ref, o_ref):
    ne = e_ref.shape[0]
    epad = jnp.pad(e_ref[...], ((0, ne_pad - ne), (0, 0)))
    o_ref[...] = jnp.dot(w_ref[...], epad,
                         preferred_element_type=jnp.float32)


def kernel(cards, mask, embedding):
    B, L = cards.shape
    NE, D = embedding.shape
    info = plsc.get_sparse_core_info()

    packed = jnp.concatenate(
        [cards.astype(jnp.int32).T,
         lax.bitcast_convert_type(mask, jnp.int32).T], axis=0)

    hist = _make_hist(B, L, info.num_cores, info.num_subcores)
    w = hist(packed)

    BM = 4096
    out = pl.pallas_call(
        functools.partial(_mm_body, NE_PAD),
        grid=(B // BM,),
        in_specs=[
            pl.BlockSpec((BM, NE_PAD), lambda i: (i, 0)),
            pl.BlockSpec((NE, D), lambda i: (0, 0)),
        ],
        out_specs=pl.BlockSpec((BM, D), lambda i: (i, 0)),
        out_shape=jax.ShapeDtypeStruct((B, D), jnp.float32),
    )(w, embedding)
    return out
